# Initial kernel scaffold; baseline (speedup 1.0000x reference)
import jax
import jax.numpy as jnp
from jax.experimental import pallas as pl


def kernel(x_cats, W):
    raise NotImplementedError("write your pallas kernel here")


# speedup vs baseline: 3.8694x; 3.8694x over previous
import functools

import jax
import jax.numpy as jnp
from jax import lax
from jax.experimental import pallas as pl
from jax.experimental.pallas import tpu as pltpu
from jax.experimental.pallas import tpu_sc as plsc

N_FIELDS = 100
VOCAB = 1000
BATCH = 16384
PROJ_DIM = 31
PAD_DIM = 32
LANES = 16

TOTAL_ROWS = BATCH * N_FIELDS
NUM_WORKERS = 32
ROWS_PER_WORKER = TOTAL_ROWS // NUM_WORKERS
CHUNK = 1024
STREAM_ROWS = 128
N_STREAMS = CHUNK // STREAM_ROWS
N_CHUNKS = ROWS_PER_WORKER // CHUNK


def _make_kernel():
    mesh = plsc.VectorSubcoreMesh(core_axis_name="c", subcore_axis_name="s")

    @functools.partial(
        pl.kernel,
        mesh=mesh,
        out_type=jax.ShapeDtypeStruct((TOTAL_ROWS * PROJ_DIM,), jnp.float32),
        scratch_types=[
            pltpu.VMEM((CHUNK,), jnp.int32),
            pltpu.VMEM((CHUNK, PAD_DIM), jnp.float32),
            pltpu.VMEM((CHUNK * PROJ_DIM,), jnp.float32),
            pltpu.SemaphoreType.DMA,
        ],
        compiler_params=pltpu.CompilerParams(use_tc_tiling_on_sc=False),
    )
    def emb_kernel(w_hbm, idx_hbm, out_hbm, idx_v, rows_v, flat_v, sem):
        nc = 2
        wid = lax.axis_index("s") * nc + lax.axis_index("c")
        w_base = wid * ROWS_PER_WORKER
        lane = lax.iota(jnp.int32, LANES)

        def chunk_body(ci, carry):
            base = w_base + ci * CHUNK
            pltpu.sync_copy(idx_hbm.at[pl.ds(base, CHUNK)], idx_v)

            def fix_body(j, c):
                off = j * LANES
                p = base + off + lane
                fld = lax.rem(p, N_FIELDS)
                idx_v[pl.ds(off, LANES)] = (
                    idx_v[pl.ds(off, LANES)] + fld * VOCAB
                )
                return c

            lax.fori_loop(0, CHUNK // LANES, fix_body, 0)

            copies = []
            for k in range(N_STREAMS):
                s = k * STREAM_ROWS
                copies.append(
                    pltpu.async_copy(
                        w_hbm.at[idx_v.at[pl.ds(s, STREAM_ROWS)]],
                        rows_v.at[pl.ds(s, STREAM_ROWS)],
                        sem,
                    )
                )
            for c in copies:
                c.wait()

            def compact(r, c):
                flat_v[pl.ds(r * PROJ_DIM, LANES)] = rows_v[r, pl.ds(0, LANES)]
                flat_v[pl.ds(r * PROJ_DIM + 15, LANES)] = (
                    rows_v[r, pl.ds(15, LANES)]
                )
                return c

            lax.fori_loop(0, CHUNK, compact, 0)

            pltpu.sync_copy(
                flat_v, out_hbm.at[pl.ds(base * PROJ_DIM, CHUNK * PROJ_DIM)]
            )
            return carry

        lax.fori_loop(0, N_CHUNKS, chunk_body, 0)

    return emb_kernel


_EMB_KERNEL = _make_kernel()


@jax.jit
def kernel(x_cats, W):
    w_pad = jnp.pad(W, ((0, 0), (0, 0), (0, PAD_DIM - PROJ_DIM)))
    w_flat = w_pad.reshape(N_FIELDS * VOCAB, PAD_DIM)
    x_flat = x_cats.astype(jnp.int32).reshape(TOTAL_ROWS)
    out = _EMB_KERNEL(w_flat, x_flat)
    return out.reshape(BATCH, N_FIELDS, PROJ_DIM)


# speedup vs baseline: 4.2092x; 1.0878x over previous
import functools

import jax
import jax.numpy as jnp
from jax import lax
from jax.experimental import pallas as pl
from jax.experimental.pallas import tpu as pltpu
from jax.experimental.pallas import tpu_sc as plsc

N_FIELDS = 100
VOCAB = 1000
BATCH = 16384
PROJ_DIM = 31
PAD_DIM = 32
LANES = 16

TOTAL_ROWS = BATCH * N_FIELDS
NUM_WORKERS = 32
ROWS_PER_WORKER = TOTAL_ROWS // NUM_WORKERS
CHUNK = 1024
STREAM_ROWS = 128
N_STREAMS = CHUNK // STREAM_ROWS
N_CHUNKS = ROWS_PER_WORKER // CHUNK
FIX_UNROLL = 4
COMPACT_UNROLL = 4


def _make_kernel():
    mesh = plsc.VectorSubcoreMesh(core_axis_name="c", subcore_axis_name="s")

    @functools.partial(
        pl.kernel,
        mesh=mesh,
        out_type=jax.ShapeDtypeStruct((TOTAL_ROWS * PROJ_DIM,), jnp.float32),
        scratch_types=[
            pltpu.VMEM((2, CHUNK), jnp.int32),
            pltpu.VMEM((2, CHUNK, PAD_DIM), jnp.float32),
            pltpu.VMEM((2, CHUNK * PROJ_DIM), jnp.float32),
            pltpu.SemaphoreType.DMA,
            pltpu.SemaphoreType.DMA,
            pltpu.SemaphoreType.DMA,
            pltpu.SemaphoreType.DMA,
            pltpu.SemaphoreType.DMA,
            pltpu.SemaphoreType.DMA,
            pltpu.SemaphoreType.DMA,
            pltpu.SemaphoreType.DMA,
            pltpu.SemaphoreType.DMA,
            pltpu.SemaphoreType.DMA,
        ],
        compiler_params=pltpu.CompilerParams(use_tc_tiling_on_sc=False),
    )
    def emb_kernel(w_hbm, idx_hbm, out_hbm, idx_v, rows_v, flat_v,
                   sem_idx, sem_out, *gsems):
        nc = 2
        wid = lax.axis_index("s") * nc + lax.axis_index("c")
        w_base = wid * ROWS_PER_WORKER
        lane = lax.iota(jnp.int32, LANES)

        def fixup(b, base):
            def fix_body(j, c):
                for u in range(FIX_UNROLL):
                    off = (j * FIX_UNROLL + u) * LANES
                    p = base + off + lane
                    fld = lax.rem(p, N_FIELDS)
                    idx_v[b, pl.ds(off, LANES)] = (
                        idx_v[b, pl.ds(off, LANES)] + fld * VOCAB
                    )
                return c

            lax.fori_loop(0, CHUNK // (LANES * FIX_UNROLL), fix_body, 0)

        def fire_gathers(b):
            copies = []
            for k in range(N_STREAMS):
                s = k * STREAM_ROWS
                copies.append(
                    pltpu.async_copy(
                        w_hbm.at[idx_v.at[b, pl.ds(s, STREAM_ROWS)]],
                        rows_v.at[b, pl.ds(s, STREAM_ROWS)],
                        gsems[k],
                    )
                )
            return copies

        def compact_all(b, copies):
            for k in range(N_STREAMS):
                copies[k].wait()
                s = k * STREAM_ROWS

                def compact(j, c, s=s):
                    for u in range(COMPACT_UNROLL):
                        r = s + j * COMPACT_UNROLL + u
                        flat_v[b, pl.ds(r * PROJ_DIM, LANES)] = (
                            rows_v[b, r, pl.ds(0, LANES)]
                        )
                        flat_v[b, pl.ds(r * PROJ_DIM + 15, LANES)] = (
                            rows_v[b, r, pl.ds(15, LANES)]
                        )
                    return c

                lax.fori_loop(0, STREAM_ROWS // COMPACT_UNROLL, compact, 0)

        def prefetch_idx(b, ci):
            nxt = lax.min(ci + 1, N_CHUNKS - 1)
            base = w_base + nxt * CHUNK
            return pltpu.async_copy(
                idx_hbm.at[pl.ds(base, CHUNK)], idx_v.at[1 - b], sem_idx
            )

        def fire_out(b, base):
            return pltpu.async_copy(
                flat_v.at[b],
                out_hbm.at[pl.ds(base * PROJ_DIM, CHUNK * PROJ_DIM)],
                sem_out,
            )

        def wait_idx(b):
            pltpu.make_async_copy(
                idx_hbm.at[pl.ds(0, CHUNK)], idx_v.at[b], sem_idx
            ).wait()

        def wait_out(b):
            pltpu.make_async_copy(
                out_hbm.at[pl.ds(0, CHUNK * PROJ_DIM)], flat_v.at[b], sem_out
            ).wait()

        pltpu.sync_copy(idx_hbm.at[pl.ds(w_base, CHUNK)], idx_v.at[0])
        fixup(0, w_base)
        copies = fire_gathers(0)
        prefetch_idx(0, 0)
        compact_all(0, copies)
        fire_out(0, w_base)

        def chunk_body(ci, carry):
            b = lax.rem(ci, 2)
            base = w_base + ci * CHUNK
            wait_idx(b)
            fixup(b, base)
            copies = fire_gathers(b)
            prefetch_idx(b, ci)
            wait_out(b)
            compact_all(b, copies)
            fire_out(b, base)
            return carry

        lax.fori_loop(1, N_CHUNKS, chunk_body, 0)

        wait_idx(0)
        wait_out(0)

    return emb_kernel


_EMB_KERNEL = _make_kernel()


@jax.jit
def kernel(x_cats, W):
    w_pad = jnp.pad(W, ((0, 0), (0, 0), (0, PAD_DIM - PROJ_DIM)))
    w_flat = w_pad.reshape(N_FIELDS * VOCAB, PAD_DIM)
    x_flat = x_cats.astype(jnp.int32).reshape(TOTAL_ROWS)
    out = _EMB_KERNEL(w_flat, x_flat)
    return out.reshape(BATCH, N_FIELDS, PROJ_DIM)


# speedup vs baseline: 5.0048x; 1.1890x over previous
import functools

import jax
import jax.numpy as jnp
from jax import lax
from jax.experimental import pallas as pl
from jax.experimental.pallas import tpu as pltpu
from jax.experimental.pallas import tpu_sc as plsc

N_FIELDS = 100
F_PAD = 104
VOCAB = 1000
BATCH = 16384
PROJ_DIM = 31
LANE_PAD = 128
LANES = 16

NUM_WORKERS = 32
B_PER_WORKER = BATCH // NUM_WORKERS
NB = 2
N_STEPS = B_PER_WORKER // NB
STEP_IDX = NB * F_PAD
COMPACT_UNROLL = 4


def _make_kernel():
    mesh = plsc.VectorSubcoreMesh(core_axis_name="c", subcore_axis_name="s")

    @functools.partial(
        pl.kernel,
        mesh=mesh,
        out_type=jax.ShapeDtypeStruct((BATCH, N_FIELDS, PROJ_DIM), jnp.float32),
        scratch_types=[
            pltpu.VMEM((2 * STEP_IDX,), jnp.int32),
            pltpu.VMEM((2, NB, N_FIELDS, LANE_PAD), jnp.float32),
            pltpu.VMEM((2, NB, N_FIELDS, PROJ_DIM), jnp.float32),
            pltpu.SemaphoreType.DMA,
            pltpu.SemaphoreType.DMA,
            pltpu.SemaphoreType.DMA,
            pltpu.SemaphoreType.DMA,
        ],
        compiler_params=pltpu.CompilerParams(use_tc_tiling_on_sc=True),
    )
    def emb_kernel(w_hbm, idx_hbm, out_hbm, idx_v, wide_v, slab_v,
                   sem_idx, sem_out, *gsems):
        nc = 2
        wid = lax.axis_index("s") * nc + lax.axis_index("c")
        w_b0 = wid * B_PER_WORKER
        lane = lax.iota(jnp.int32, LANES)

        def fixup(b):
            for g in range(STEP_IDX // LANES):
                off = g * LANES
                fld = lax.rem(off + lane, F_PAD)
                q = b * STEP_IDX + off
                idx_v[pl.ds(q, LANES)] = (
                    idx_v[pl.ds(q, LANES)] + fld * VOCAB
                )

        def fire_gathers(b):
            copies = []
            for i in range(NB):
                q = b * STEP_IDX + i * F_PAD
                copies.append(
                    pltpu.async_copy(
                        w_hbm.at[idx_v.at[pl.ds(q, N_FIELDS)]],
                        wide_v.at[b, i],
                        gsems[i],
                    )
                )
            return copies

        def compact(b, i):
            def body(j, c):
                for u in range(COMPACT_UNROLL):
                    f = j * COMPACT_UNROLL + u
                    slab_v[b, i, f, pl.ds(0, LANES)] = (
                        wide_v[b, i, f, pl.ds(0, LANES)]
                    )
                    slab_v[b, i, f, pl.ds(15, LANES)] = (
                        wide_v[b, i, f, pl.ds(15, LANES)]
                    )
                return c

            lax.fori_loop(0, N_FIELDS // COMPACT_UNROLL, body, 0)

        def prefetch_idx(b, si):
            nxt = lax.min(si + 1, N_STEPS - 1)
            base = (w_b0 + nxt * NB) * F_PAD
            return pltpu.async_copy(
                idx_hbm.at[pl.ds(base, STEP_IDX)],
                idx_v.at[pl.ds((1 - b) * STEP_IDX, STEP_IDX)],
                sem_idx,
            )

        def fire_out(b, b0):
            return pltpu.async_copy(
                slab_v.at[b], out_hbm.at[pl.ds(b0, NB)], sem_out
            )

        def wait_idx():
            pltpu.make_async_copy(
                idx_hbm.at[pl.ds(0, STEP_IDX)],
                idx_v.at[pl.ds(0, STEP_IDX)],
                sem_idx,
            ).wait()

        def wait_out(b):
            pltpu.make_async_copy(
                out_hbm.at[pl.ds(0, NB)], slab_v.at[b], sem_out
            ).wait()

        pltpu.sync_copy(
            idx_hbm.at[pl.ds(w_b0 * F_PAD, STEP_IDX)],
            idx_v.at[pl.ds(0, STEP_IDX)],
        )
        fixup(0)
        copies = fire_gathers(0)
        prefetch_idx(0, 0)
        for i in range(NB):
            copies[i].wait()
            compact(0, i)
        fire_out(0, w_b0)

        def step_body(si, carry):
            b = lax.rem(si, 2)
            wait_idx()
            fixup(b)
            copies = fire_gathers(b)
            prefetch_idx(b, si)
            wait_out(b)
            for i in range(NB):
                copies[i].wait()
                compact(b, i)
            fire_out(b, w_b0 + si * NB)
            return carry

        lax.fori_loop(1, N_STEPS, step_body, 0)

        wait_idx()
        wait_out(0)

    return emb_kernel


_EMB_KERNEL = _make_kernel()


@jax.jit
def kernel(x_cats, W):
    w_pad = jnp.pad(W, ((0, 0), (0, 0), (0, LANE_PAD - PROJ_DIM)))
    w_flat = w_pad.reshape(N_FIELDS * VOCAB, LANE_PAD)
    x_pad = jnp.pad(x_cats.astype(jnp.int32), ((0, 0), (0, F_PAD - N_FIELDS)))
    x_flat = x_pad.reshape(BATCH * F_PAD)
    return _EMB_KERNEL(w_flat, x_flat)


# speedup vs baseline: 6.0376x; 1.2064x over previous
import functools

import jax
import jax.numpy as jnp
from jax import lax
from jax.experimental import pallas as pl
from jax.experimental.pallas import tpu as pltpu
from jax.experimental.pallas import tpu_sc as plsc

N_FIELDS = 100
F_PAD = 104
VOCAB = 1000
BATCH = 16384
PROJ_DIM = 31
LANE_PAD = 128
LANES = 16

NUM_WORKERS = 32
BC = BATCH // NUM_WORKERS
N_OCT = F_PAD // 8
STREAM_ROWS = 128
N_STREAMS = BC // STREAM_ROWS


def _make_kernel():
    mesh = plsc.VectorSubcoreMesh(core_axis_name="c", subcore_axis_name="s")

    @functools.partial(
        pl.kernel,
        mesh=mesh,
        out_type=jax.ShapeDtypeStruct((N_FIELDS, PROJ_DIM, BATCH), jnp.float32),
        scratch_types=[
            pltpu.VMEM((2, 8, BC), jnp.int32),
            pltpu.VMEM((BC,), jnp.int32),
            pltpu.VMEM((2, STREAM_ROWS, LANE_PAD), jnp.float32),
            pltpu.VMEM((2, PROJ_DIM, BC), jnp.float32),
            pltpu.SemaphoreType.DMA,
            pltpu.SemaphoreType.DMA,
            pltpu.SemaphoreType.DMA,
            pltpu.SemaphoreType.DMA,
        ],
        compiler_params=pltpu.CompilerParams(use_tc_tiling_on_sc=True, needs_layout_passes=False),
    )
    def emb_kernel(w_hbm, xt_hbm, out_hbm, xst_v, idx_v, sbuf_v, slab_v,
                   sem_idx, sem_out, *gsems):
        nc = 2
        wid = lax.axis_index("s") * nc + lax.axis_index("c")
        b0 = pl.multiple_of(wid * BC, LANE_PAD)
        lane = lax.iota(jnp.int32, LANES)
        zeros = lane * 0

        def load_oct(buf, o):
            f0 = pl.multiple_of(o * 8, 8)
            return pltpu.async_copy(
                xt_hbm.at[pl.ds(f0, 8), pl.ds(b0, BC)],
                xst_v.at[buf], sem_idx,
            )

        def wait_oct(buf):
            pltpu.make_async_copy(
                xt_hbm.at[pl.ds(0, 8), pl.ds(0, BC)], xst_v.at[buf], sem_idx
            ).wait()

        def fixup(buf, f_loc, f):
            off = f * VOCAB
            for g in range(BC // LANES):
                idx_v[pl.ds(g * LANES, LANES)] = (
                    xst_v[buf, f_loc, pl.ds(g * LANES, LANES)] + off
                )

        def transpose_rows(r, sb, sbase):
            def body(j, c):
                for u in range(4):
                    k = j * 4 + u
                    col = sbase + k
                    lo = sbuf_v[r, k, pl.ds(0, LANES)]
                    hi = sbuf_v[r, k, pl.ds(15, LANES)]
                    bcol = zeros + col
                    plsc.store_scatter(
                        slab_v.at[sb], [lane, bcol], lo
                    )
                    plsc.store_scatter(
                        slab_v.at[sb], [lane + 15, bcol], hi
                    )
                return c

            lax.fori_loop(0, STREAM_ROWS // 4, body, 0)

        def fire_stream(r, s):
            return pltpu.async_copy(
                w_hbm.at[idx_v.at[pl.ds(s * STREAM_ROWS, STREAM_ROWS)]],
                sbuf_v.at[r], gsems[r],
            )

        def fire_out(sb, f):
            return pltpu.async_copy(
                slab_v.at[sb], out_hbm.at[f, pl.ds(0, PROJ_DIM), pl.ds(b0, BC)],
                sem_out,
            )

        def wait_out(sb):
            pltpu.make_async_copy(
                out_hbm.at[0, pl.ds(0, PROJ_DIM), pl.ds(0, BC)],
                slab_v.at[sb], sem_out,
            ).wait()

        def do_field(buf, f_loc, f, sb, first):
            fixup(buf, f_loc, f)
            c0 = fire_stream(0, 0)
            c1 = fire_stream(1, 1)
            if not first:
                wait_out(sb)
            c0.wait()
            transpose_rows(0, sb, 0)
            c2 = fire_stream(0, 2)
            c1.wait()
            transpose_rows(1, sb, STREAM_ROWS)
            c3 = fire_stream(1, 3)
            c2.wait()
            transpose_rows(0, sb, 2 * STREAM_ROWS)
            c3.wait()
            transpose_rows(1, sb, 3 * STREAM_ROWS)
            fire_out(sb, f)

        load_oct(0, 0).wait()
        load_oct(1, 1)
        for f_loc in range(8):
            do_field(0, f_loc, f_loc, f_loc % 2, first=f_loc < 2)

        def oct_body(o, carry):
            buf = lax.rem(o, 2)
            wait_oct(buf)
            nxt = lax.min(o + 1, N_OCT - 1)
            load_oct(1 - buf, nxt)
            for f_loc in range(8):
                f = o * 8 + f_loc

                @pl.when(f < N_FIELDS)
                def _():
                    do_field(buf, f_loc, f, f_loc % 2, first=False)

            return carry

        lax.fori_loop(1, N_OCT, oct_body, 0)

        wait_oct(0)
        wait_out(0)
        wait_out(1)

    return emb_kernel


_EMB_KERNEL = _make_kernel()


@jax.jit
def kernel(x_cats, W):
    w_pad = jnp.pad(W, ((0, 0), (0, 0), (0, LANE_PAD - PROJ_DIM)))
    w_flat = w_pad.reshape(N_FIELDS * VOCAB, LANE_PAD)
    xt = jnp.transpose(x_cats.astype(jnp.int32))
    xt_pad = jnp.pad(xt, ((0, F_PAD - N_FIELDS), (0, 0)))
    raw = _EMB_KERNEL(w_flat, xt_pad)
    return jnp.transpose(raw, (2, 0, 1))


# speedup vs baseline: 6.9187x; 1.1459x over previous
import functools

import jax
import jax.numpy as jnp
from jax import lax
from jax.experimental import pallas as pl
from jax.experimental.pallas import tpu as pltpu
from jax.experimental.pallas import tpu_sc as plsc

N_FIELDS = 100
F_PAD = 104
VOCAB = 1000
BATCH = 16384
PROJ_DIM = 31
LANE_PAD = 128
LANES = 16

NUM_WORKERS = 32
BC = BATCH // NUM_WORKERS
N_OCT = F_PAD // 8
STREAM_ROWS = 128
N_STREAMS = BC // STREAM_ROWS


def _make_kernel():
    mesh = plsc.VectorSubcoreMesh(core_axis_name="c", subcore_axis_name="s")

    @functools.partial(
        pl.kernel,
        mesh=mesh,
        out_type=jax.ShapeDtypeStruct((N_FIELDS, PROJ_DIM, BATCH), jnp.float32),
        scratch_types=[
            pltpu.VMEM((2, 8, BC), jnp.int32),
            pltpu.VMEM((BC,), jnp.int32),
            pltpu.VMEM((2, STREAM_ROWS, LANE_PAD), jnp.float32),
            pltpu.VMEM((2, PROJ_DIM, BC), jnp.float32),
            pltpu.SemaphoreType.DMA,
            pltpu.SemaphoreType.DMA,
            pltpu.SemaphoreType.DMA,
            pltpu.SemaphoreType.DMA,
        ],
        compiler_params=pltpu.CompilerParams(use_tc_tiling_on_sc=True, needs_layout_passes=False),
    )
    def emb_kernel(w_hbm, xt_hbm, out_hbm, xst_v, idx_v, sbuf_v, slab_v,
                   sem_idx, sem_out, *gsems):
        nc = 2
        wid = lax.axis_index("s") * nc + lax.axis_index("c")
        b0 = pl.multiple_of(wid * BC, LANE_PAD)
        lane = lax.iota(jnp.int32, LANES)
        zeros = lane * 0

        def load_oct(buf, o):
            f0 = pl.multiple_of(o * 8, 8)
            return pltpu.async_copy(
                xt_hbm.at[pl.ds(f0, 8), pl.ds(b0, BC)],
                xst_v.at[buf], sem_idx,
            )

        def wait_oct(buf):
            pltpu.make_async_copy(
                xt_hbm.at[pl.ds(0, 8), pl.ds(0, BC)], xst_v.at[buf], sem_idx
            ).wait()

        def fixup(buf, f_loc, f):
            off = f * VOCAB
            for g in range(BC // LANES):
                idx_v[pl.ds(g * LANES, LANES)] = (
                    xst_v[buf, f_loc, pl.ds(g * LANES, LANES)] + off
                )

        def transpose_rows(r, sb, sbase):
            @plsc.parallel_loop(0, STREAM_ROWS, unroll=8)
            def _(k):
                col = sbase + k
                lo = sbuf_v[r, k, pl.ds(0, LANES)]
                hi = sbuf_v[r, k, pl.ds(15, LANES)]
                bcol = zeros + col
                plsc.store_scatter(slab_v.at[sb], [lane, bcol], lo)
                plsc.store_scatter(slab_v.at[sb], [lane + 15, bcol], hi)

        def fire_stream(r, s):
            return pltpu.async_copy(
                w_hbm.at[idx_v.at[pl.ds(s * STREAM_ROWS, STREAM_ROWS)]],
                sbuf_v.at[r], gsems[r],
            )

        def fire_out(sb, f):
            return pltpu.async_copy(
                slab_v.at[sb], out_hbm.at[f, pl.ds(0, PROJ_DIM), pl.ds(b0, BC)],
                sem_out,
            )

        def wait_out(sb):
            pltpu.make_async_copy(
                out_hbm.at[0, pl.ds(0, PROJ_DIM), pl.ds(0, BC)],
                slab_v.at[sb], sem_out,
            ).wait()

        def do_field(buf, f_loc, f, sb, first):
            fixup(buf, f_loc, f)
            c0 = fire_stream(0, 0)
            c1 = fire_stream(1, 1)
            if not first:
                wait_out(sb)
            c0.wait()
            transpose_rows(0, sb, 0)
            c2 = fire_stream(0, 2)
            c1.wait()
            transpose_rows(1, sb, STREAM_ROWS)
            c3 = fire_stream(1, 3)
            c2.wait()
            transpose_rows(0, sb, 2 * STREAM_ROWS)
            c3.wait()
            transpose_rows(1, sb, 3 * STREAM_ROWS)
            fire_out(sb, f)

        load_oct(0, 0).wait()
        load_oct(1, 1)
        for f_loc in range(8):
            do_field(0, f_loc, f_loc, f_loc % 2, first=f_loc < 2)

        def oct_body(o, carry):
            buf = lax.rem(o, 2)
            wait_oct(buf)
            nxt = lax.min(o + 1, N_OCT - 1)
            load_oct(1 - buf, nxt)
            for f_loc in range(8):
                f = o * 8 + f_loc

                @pl.when(f < N_FIELDS)
                def _():
                    do_field(buf, f_loc, f, f_loc % 2, first=False)

            return carry

        lax.fori_loop(1, N_OCT, oct_body, 0)

        wait_oct(0)
        wait_out(0)
        wait_out(1)

    return emb_kernel


_EMB_KERNEL = _make_kernel()


@jax.jit
def kernel(x_cats, W):
    w_pad = jnp.pad(W, ((0, 0), (0, 0), (0, LANE_PAD - PROJ_DIM)))
    w_flat = w_pad.reshape(N_FIELDS * VOCAB, LANE_PAD)
    xt = jnp.transpose(x_cats.astype(jnp.int32))
    xt_pad = jnp.pad(xt, ((0, F_PAD - N_FIELDS), (0, 0)))
    raw = _EMB_KERNEL(w_flat, xt_pad)
    return jnp.transpose(raw, (2, 0, 1))


# speedup vs baseline: 9.2108x; 1.3313x over previous
import functools

import jax
import jax.numpy as jnp
from jax import lax
from jax.experimental import pallas as pl
from jax.experimental.pallas import tpu as pltpu
from jax.experimental.pallas import tpu_sc as plsc

N_FIELDS = 100
F_PAD = 104
VOCAB = 1000
BATCH = 16384
PROJ_DIM = 31
LANE_PAD = 128
LANES = 16

NUM_WORKERS = 32
BC = BATCH // NUM_WORKERS
N_OCT = F_PAD // 8
STREAM_ROWS = 128
N_STREAMS = BC // STREAM_ROWS


def _make_kernel():
    mesh = plsc.VectorSubcoreMesh(core_axis_name="c", subcore_axis_name="s")

    @functools.partial(
        pl.kernel,
        mesh=mesh,
        out_type=jax.ShapeDtypeStruct((N_FIELDS, PROJ_DIM, BATCH), jnp.float32),
        scratch_types=[
            pltpu.VMEM((2, 8, BC), jnp.int32),
            pltpu.VMEM((BC,), jnp.int32),
            pltpu.VMEM((2, STREAM_ROWS, LANE_PAD), jnp.float32),
            pltpu.VMEM((2, PROJ_DIM, BC), jnp.float32),
            pltpu.SemaphoreType.DMA,
            pltpu.SemaphoreType.DMA,
            pltpu.SemaphoreType.DMA,
            pltpu.SemaphoreType.DMA,
        ],
        compiler_params=pltpu.CompilerParams(use_tc_tiling_on_sc=True, needs_layout_passes=False),
    )
    def emb_kernel(w_hbm, xt_hbm, out_hbm, xst_v, idx_v, sbuf_v, slab_v,
                   sem_idx, sem_out, *gsems):
        nc = 2
        wid = lax.axis_index("s") * nc + lax.axis_index("c")
        b0 = pl.multiple_of(wid * BC, LANE_PAD)
        lane = lax.iota(jnp.int32, LANES)
        zeros = lane * 0

        def load_oct(buf, o):
            f0 = pl.multiple_of(o * 8, 8)
            return pltpu.async_copy(
                xt_hbm.at[pl.ds(f0, 8), pl.ds(b0, BC)],
                xst_v.at[buf], sem_idx,
            )

        def wait_oct(buf):
            pltpu.make_async_copy(
                xt_hbm.at[pl.ds(0, 8), pl.ds(0, BC)], xst_v.at[buf], sem_idx
            ).wait()

        def fixup(buf, f_loc, f):
            off = f * VOCAB
            for g in range(BC // LANES):
                idx_v[pl.ds(g * LANES, LANES)] = (
                    xst_v[buf, f_loc, pl.ds(g * LANES, LANES)] + off
                )

        def transpose_rows(r, sb, sbase):
            @plsc.parallel_loop(0, STREAM_ROWS // LANES, unroll=2)
            def _(cb):
                rowbase = cb * LANES
                row_idx = lane + rowbase
                col_idx = lane + (sbase + rowbase)
                for p in range(LANES):
                    dlo = lax.rem(lane + p, LANES)
                    lo = plsc.load_gather(sbuf_v.at[r], [row_idx, dlo])
                    plsc.store_scatter(slab_v.at[sb], [dlo, col_idx], lo)
                    dhi = dlo + 15
                    hi = plsc.load_gather(sbuf_v.at[r], [row_idx, dhi])
                    plsc.store_scatter(slab_v.at[sb], [dhi, col_idx], hi)

        def fire_stream(r, s):
            return pltpu.async_copy(
                w_hbm.at[idx_v.at[pl.ds(s * STREAM_ROWS, STREAM_ROWS)]],
                sbuf_v.at[r], gsems[r],
            )

        def fire_out(sb, f):
            return pltpu.async_copy(
                slab_v.at[sb], out_hbm.at[f, pl.ds(0, PROJ_DIM), pl.ds(b0, BC)],
                sem_out,
            )

        def wait_out(sb):
            pltpu.make_async_copy(
                out_hbm.at[0, pl.ds(0, PROJ_DIM), pl.ds(0, BC)],
                slab_v.at[sb], sem_out,
            ).wait()

        def do_field(buf, f_loc, f, sb, not_first):
            fixup(buf, f_loc, f)
            c0 = fire_stream(0, 0)
            c1 = fire_stream(1, 1)

            @pl.when(not_first)
            def _():
                wait_out(sb)

            c0.wait()
            transpose_rows(0, sb, 0)
            c2 = fire_stream(0, 2)
            c1.wait()
            transpose_rows(1, sb, STREAM_ROWS)
            c3 = fire_stream(1, 3)
            c2.wait()
            transpose_rows(0, sb, 2 * STREAM_ROWS)
            c3.wait()
            transpose_rows(1, sb, 3 * STREAM_ROWS)
            fire_out(sb, f)

        pltpu.sync_copy(
            xt_hbm.at[pl.ds(0, 8), pl.ds(b0, BC)], xst_v.at[0]
        )

        def oct_body(o, carry):
            buf = lax.rem(o, 2)

            @pl.when(o > 0)
            def _():
                wait_oct(buf)

            @pl.when(o < N_OCT - 1)
            def _():
                load_oct(1 - buf, o + 1)

            def field_body(f_loc, c):
                f = o * 8 + f_loc
                sb = lax.rem(f_loc, 2)
                not_first = jnp.logical_or(o > 0, f_loc >= 2)

                @pl.when(f < N_FIELDS)
                def _():
                    do_field(buf, f_loc, f, sb, not_first)

                return c

            lax.fori_loop(0, 8, field_body, 0)
            return carry

        lax.fori_loop(0, N_OCT, oct_body, 0)

        wait_out(0)
        wait_out(1)

    return emb_kernel


_EMB_KERNEL = _make_kernel()


@jax.jit
def kernel(x_cats, W):
    w_pad = jnp.pad(W, ((0, 0), (0, 0), (0, LANE_PAD - PROJ_DIM)))
    w_flat = w_pad.reshape(N_FIELDS * VOCAB, LANE_PAD)
    xt = jnp.transpose(x_cats.astype(jnp.int32))
    xt_pad = jnp.pad(xt, ((0, F_PAD - N_FIELDS), (0, 0)))
    raw = _EMB_KERNEL(w_flat, xt_pad)
    return jnp.transpose(raw, (2, 0, 1))


# speedup vs baseline: 12.5829x; 1.3661x over previous
import functools

import jax
import jax.numpy as jnp
from jax import lax
from jax.experimental import pallas as pl
from jax.experimental.pallas import tpu as pltpu
from jax.experimental.pallas import tpu_sc as plsc

N_FIELDS = 100
F_PAD = 104
VOCAB = 1000
BATCH = 16384
PROJ_DIM = 31
LANE_PAD = 128
LANES = 16

NUM_WORKERS = 32
BC = BATCH // NUM_WORKERS
N_OCT = F_PAD // 8
STREAM_ROWS = 128
N_STREAMS = BC // STREAM_ROWS


def _make_kernel():
    mesh = plsc.VectorSubcoreMesh(core_axis_name="c", subcore_axis_name="s")

    @functools.partial(
        pl.kernel,
        mesh=mesh,
        out_type=jax.ShapeDtypeStruct((N_FIELDS, PROJ_DIM, BATCH), jnp.float32),
        scratch_types=[
            pltpu.VMEM((2, 8, BC), jnp.int32),
            pltpu.VMEM((BC,), jnp.int32),
            pltpu.VMEM((2, STREAM_ROWS, LANE_PAD), jnp.float32),
            pltpu.VMEM((2, PROJ_DIM, BC), jnp.float32),
            pltpu.SemaphoreType.DMA,
            pltpu.SemaphoreType.DMA,
            pltpu.SemaphoreType.DMA,
            pltpu.SemaphoreType.DMA,
        ],
        compiler_params=pltpu.CompilerParams(use_tc_tiling_on_sc=True, needs_layout_passes=False),
    )
    def emb_kernel(w_hbm, xt_hbm, out_hbm, xst_v, idx_v, sbuf_v, slab_v,
                   sem_idx, sem_out, *gsems):
        nc = 2
        wid = lax.axis_index("s") * nc + lax.axis_index("c")
        b0 = pl.multiple_of(wid * BC, LANE_PAD)
        lane = lax.iota(jnp.int32, LANES)
        zeros = lane * 0

        def load_oct(buf, o):
            f0 = pl.multiple_of(o * 8, 8)
            return pltpu.async_copy(
                xt_hbm.at[pl.ds(f0, 8), pl.ds(b0, BC)],
                xst_v.at[buf], sem_idx,
            )

        def wait_oct(buf):
            pltpu.make_async_copy(
                xt_hbm.at[pl.ds(0, 8), pl.ds(0, BC)], xst_v.at[buf], sem_idx
            ).wait()

        def fixup(buf, f_loc, f):
            off = f * VOCAB
            for g in range(BC // LANES):
                idx_v[pl.ds(g * LANES, LANES)] = (
                    xst_v[buf, f_loc, pl.ds(g * LANES, LANES)] + off
                )

        def transpose_rows(r, sb, sbase):
            @plsc.parallel_loop(0, STREAM_ROWS // LANES, unroll=4)
            def _(cb):
                rowbase = cb * LANES
                row_idx = lane + rowbase
                col_idx = lane + (sbase + rowbase)
                for p in range(LANES):
                    dlo = (lane + p) & (LANES - 1)
                    lo = plsc.load_gather(sbuf_v.at[r], [row_idx, dlo])
                    plsc.store_scatter(slab_v.at[sb], [dlo, col_idx], lo)
                    dhi = dlo + 15
                    hi = plsc.load_gather(sbuf_v.at[r], [row_idx, dhi])
                    plsc.store_scatter(slab_v.at[sb], [dhi, col_idx], hi)

        def fire_stream(r, s):
            return pltpu.async_copy(
                w_hbm.at[idx_v.at[pl.ds(s * STREAM_ROWS, STREAM_ROWS)]],
                sbuf_v.at[r], gsems[r],
            )

        def fire_out(sb, f):
            return pltpu.async_copy(
                slab_v.at[sb], out_hbm.at[f, pl.ds(0, PROJ_DIM), pl.ds(b0, BC)],
                sem_out,
            )

        def wait_out(sb):
            pltpu.make_async_copy(
                out_hbm.at[0, pl.ds(0, PROJ_DIM), pl.ds(0, BC)],
                slab_v.at[sb], sem_out,
            ).wait()

        def do_field(buf, f_loc, f, sb, not_first):
            fixup(buf, f_loc, f)
            c0 = fire_stream(0, 0)
            c1 = fire_stream(1, 1)

            @pl.when(not_first)
            def _():
                wait_out(sb)

            c0.wait()
            transpose_rows(0, sb, 0)
            c2 = fire_stream(0, 2)
            c1.wait()
            transpose_rows(1, sb, STREAM_ROWS)
            c3 = fire_stream(1, 3)
            c2.wait()
            transpose_rows(0, sb, 2 * STREAM_ROWS)
            c3.wait()
            transpose_rows(1, sb, 3 * STREAM_ROWS)
            fire_out(sb, f)

        pltpu.sync_copy(
            xt_hbm.at[pl.ds(0, 8), pl.ds(b0, BC)], xst_v.at[0]
        )

        def oct_body(o, carry):
            buf = lax.rem(o, 2)

            @pl.when(o > 0)
            def _():
                wait_oct(buf)

            @pl.when(o < N_OCT - 1)
            def _():
                load_oct(1 - buf, o + 1)

            def field_body(f_loc, c):
                f = o * 8 + f_loc
                sb = lax.rem(f_loc, 2)
                not_first = jnp.logical_or(o > 0, f_loc >= 2)

                @pl.when(f < N_FIELDS)
                def _():
                    do_field(buf, f_loc, f, sb, not_first)

                return c

            lax.fori_loop(0, 8, field_body, 0)
            return carry

        lax.fori_loop(0, N_OCT, oct_body, 0)

        wait_out(0)
        wait_out(1)

    return emb_kernel


_EMB_KERNEL = _make_kernel()


@jax.jit
def kernel(x_cats, W):
    w_pad = jnp.pad(W, ((0, 0), (0, 0), (0, LANE_PAD - PROJ_DIM)))
    w_flat = w_pad.reshape(N_FIELDS * VOCAB, LANE_PAD)
    xt = jnp.transpose(x_cats.astype(jnp.int32))
    xt_pad = jnp.pad(xt, ((0, F_PAD - N_FIELDS), (0, 0)))
    raw = _EMB_KERNEL(w_flat, xt_pad)
    return jnp.transpose(raw, (2, 0, 1))
